# baseline (speedup 1.0000x reference)
import functools

import jax
import jax.numpy as jnp
from jax import lax
from jax.experimental import pallas as pl
from jax.experimental.pallas import tpu as pltpu
from jax.experimental.pallas import tpu_sc as plsc

N = 10000
N_PAD = 10240
D_IN = 128
D_H = 256
HALF = 128
B_GRAPHS = 64
E = 320000
CHUNK = 128
TILES = 16
CHUNKS_PER_TILE = 160
GROUP = 32
E_PAD = TILES * CHUNKS_PER_TILE * CHUNK
ROWS_PER_TILE = N_PAD // TILES
BLK = 256
GRID = N_PAD // BLK


def _sc_aggregate(h0, h1, src, dst):
    mesh = plsc.VectorSubcoreMesh(core_axis_name="c", subcore_axis_name="s")
    out_t = (
        jax.ShapeDtypeStruct((N_PAD, HALF), jnp.float32),
        jax.ShapeDtypeStruct((N_PAD, HALF), jnp.float32),
    )

    @functools.partial(
        pl.kernel,
        mesh=mesh,
        out_type=out_t,
        scratch_types=[
            pltpu.VMEM_SHARED((N_PAD, HALF), jnp.float32),
            pltpu.VMEM((GROUP, CHUNK), jnp.int32),
            pltpu.VMEM((GROUP, CHUNK), jnp.int32),
            pltpu.VMEM((CHUNK, HALF), jnp.float32),
            pltpu.VMEM((CHUNK, HALF), jnp.float32),
            pltpu.SemaphoreType.DMA,
            pltpu.SemaphoreType.DMA,
            pltpu.SemaphoreType.DMA,
            pltpu.SemaphoreType.DMA,
        ],
    )
    def agg_kernel(h0_hbm, h1_hbm, src_hbm, dst_hbm, z0_hbm, z1_hbm,
                   acc_sh, src_v, dst_v, rows_a, rows_b,
                   gsem_a, gsem_b, ssem_a, ssem_b):
        c = lax.axis_index("c")
        s = lax.axis_index("s")
        base = s * ROWS_PER_TILE

        def run(h_hbm, z_hbm):
            pltpu.sync_copy(h_hbm.at[pl.ds(base, ROWS_PER_TILE)],
                            acc_sh.at[pl.ds(base, ROWS_PER_TILE)])
            plsc.subcore_barrier()

            @pl.loop(0, CHUNKS_PER_TILE // GROUP)
            def _(g):
                pltpu.sync_copy(src_hbm.at[s].at[pl.ds(g * GROUP, GROUP)], src_v)
                pltpu.sync_copy(dst_hbm.at[s].at[pl.ds(g * GROUP, GROUP)], dst_v)

                def wait_gather(buf, sem):
                    pltpu.make_async_copy(h_hbm.at[pl.ds(0, CHUNK)], buf, sem).wait()

                def wait_scatter(buf, sem):
                    pltpu.make_async_copy(h_hbm.at[pl.ds(0, CHUNK)], buf, sem).wait()

                pltpu.async_copy(h_hbm.at[src_v.at[0]], rows_a, gsem_a)
                pltpu.async_copy(h_hbm.at[src_v.at[1]], rows_b, gsem_b)

                @pl.loop(0, GROUP, step=2)
                def _(j):
                    wait_gather(rows_a, gsem_a)
                    pltpu.async_copy(rows_a, acc_sh.at[dst_v.at[j]], ssem_a,
                                     add=True)
                    wait_gather(rows_b, gsem_b)
                    pltpu.async_copy(rows_b, acc_sh.at[dst_v.at[j + 1]],
                                     ssem_b, add=True)

                    @pl.when(j + 2 < GROUP)
                    def _():
                        wait_scatter(rows_a, ssem_a)
                        pltpu.async_copy(h_hbm.at[src_v.at[j + 2]], rows_a, gsem_a)

                    @pl.when(j + 3 < GROUP)
                    def _():
                        wait_scatter(rows_b, ssem_b)
                        pltpu.async_copy(h_hbm.at[src_v.at[j + 3]], rows_b, gsem_b)

                wait_scatter(rows_a, ssem_a)
                wait_scatter(rows_b, ssem_b)

            plsc.subcore_barrier()
            pltpu.sync_copy(acc_sh.at[pl.ds(base, ROWS_PER_TILE)],
                            z_hbm.at[pl.ds(base, ROWS_PER_TILE)])

        @pl.when(c == 0)
        def _():
            run(h0_hbm, z0_hbm)

        @pl.when(c == 1)
        def _():
            run(h1_hbm, z1_hbm)

    return agg_kernel(h0, h1, src, dst)


def _mlp1_body(z0_ref, z1_ref, w_ref, b_ref, h1_ref, st_ref):
    i = pl.program_id(0)
    m = (
        jnp.dot(z0_ref[...], w_ref[0], preferred_element_type=jnp.float32)
        + jnp.dot(z1_ref[...], w_ref[1], preferred_element_type=jnp.float32)
        + b_ref[...]
    )
    h1_ref[...] = m
    rows = i * BLK + lax.broadcasted_iota(jnp.int32, (BLK, 1), 0)
    mm = m * (rows < N).astype(jnp.float32)
    su = jnp.sum(mm, axis=0, keepdims=True)
    sq = jnp.sum(mm * mm, axis=0, keepdims=True)
    upd = jnp.concatenate([su, sq, jnp.zeros((6, D_H), jnp.float32)], axis=0)

    @pl.when(i == 0)
    def _():
        st_ref[...] = jnp.zeros_like(st_ref)

    st_ref[...] += upd


def _mlp1(z0, z1, w3, b1):
    return pl.pallas_call(
        _mlp1_body,
        grid=(GRID,),
        in_specs=[
            pl.BlockSpec((BLK, HALF), lambda i: (i, 0)),
            pl.BlockSpec((BLK, HALF), lambda i: (i, 0)),
            pl.BlockSpec((2, HALF, D_H), lambda i: (0, 0, 0)),
            pl.BlockSpec((1, D_H), lambda i: (0, 0)),
        ],
        out_specs=[
            pl.BlockSpec((BLK, D_H), lambda i: (i, 0)),
            pl.BlockSpec((8, D_H), lambda i: (0, 0)),
        ],
        out_shape=[
            jax.ShapeDtypeStruct((N_PAD, D_H), jnp.float32),
            jax.ShapeDtypeStruct((8, D_H), jnp.float32),
        ],
    )(z0, z1, w3, b1)


def _mlp2_body(h1_ref, st_ref, g_ref, bt_ref, w2_ref, b2_ref, o0_ref, o1_ref):
    st = st_ref[...]
    mean = st[0:1, :] * (1.0 / N)
    var = st[1:2, :] * (1.0 / N) - mean * mean
    inv = lax.rsqrt(var + 1e-5)
    hn = (h1_ref[...] - mean) * (inv * g_ref[...]) + bt_ref[...]
    hn = jnp.maximum(hn, 0.0)
    out = jnp.dot(hn, w2_ref[...], preferred_element_type=jnp.float32) + b2_ref[...]
    out = jnp.maximum(out, 0.0)
    o0_ref[...] = out[:, :HALF]
    o1_ref[...] = out[:, HALF:]


def _mlp2(h1, st, gamma, beta, w2, b2):
    return pl.pallas_call(
        _mlp2_body,
        grid=(GRID,),
        in_specs=[
            pl.BlockSpec((BLK, D_H), lambda i: (i, 0)),
            pl.BlockSpec((8, D_H), lambda i: (0, 0)),
            pl.BlockSpec((1, D_H), lambda i: (0, 0)),
            pl.BlockSpec((1, D_H), lambda i: (0, 0)),
            pl.BlockSpec((D_H, D_H), lambda i: (0, 0)),
            pl.BlockSpec((1, D_H), lambda i: (0, 0)),
        ],
        out_specs=[
            pl.BlockSpec((BLK, HALF), lambda i: (i, 0)),
            pl.BlockSpec((BLK, HALF), lambda i: (i, 0)),
        ],
        out_shape=[
            jax.ShapeDtypeStruct((N_PAD, HALF), jnp.float32),
            jax.ShapeDtypeStruct((N_PAD, HALF), jnp.float32),
        ],
    )(h1, st, gamma, beta, w2, b2)


def _pool_body(h0_ref, h1_ref, p_ref, o0_ref, o1_ref):
    i = pl.program_id(0)

    @pl.when(i == 0)
    def _():
        o0_ref[...] = jnp.zeros_like(o0_ref)
        o1_ref[...] = jnp.zeros_like(o1_ref)

    pt = p_ref[...]
    dn = (((0,), (0,)), ((), ()))
    o0_ref[...] += lax.dot_general(pt, h0_ref[...], dn,
                                   preferred_element_type=jnp.float32)
    o1_ref[...] += lax.dot_general(pt, h1_ref[...], dn,
                                   preferred_element_type=jnp.float32)


def _pool(h0, h1, pmat):
    return pl.pallas_call(
        _pool_body,
        grid=(GRID,),
        in_specs=[
            pl.BlockSpec((BLK, HALF), lambda i: (i, 0)),
            pl.BlockSpec((BLK, HALF), lambda i: (i, 0)),
            pl.BlockSpec((BLK, B_GRAPHS), lambda i: (i, 0)),
        ],
        out_specs=[
            pl.BlockSpec((B_GRAPHS, HALF), lambda i: (0, 0)),
            pl.BlockSpec((B_GRAPHS, HALF), lambda i: (0, 0)),
        ],
        out_shape=[
            jax.ShapeDtypeStruct((B_GRAPHS, HALF), jnp.float32),
            jax.ShapeDtypeStruct((B_GRAPHS, HALF), jnp.float32),
        ],
    )(h0, h1, pmat)


def kernel(x, adj_t, batch, params):
    src = jnp.concatenate([adj_t[0], jnp.zeros((E_PAD - E,), jnp.int32)])
    dst = jnp.concatenate([adj_t[1], jnp.full((E_PAD - E,), N, jnp.int32)])
    src = src.reshape(TILES, CHUNKS_PER_TILE, CHUNK)
    dst = dst.reshape(TILES, CHUNKS_PER_TILE, CHUNK)

    n_idx = jnp.arange(N_PAD, dtype=jnp.int32)
    seg = jnp.clip(jnp.searchsorted(batch, n_idx, side="right") - 1, 0, B_GRAPHS - 1)
    cnt = (batch[1:] - batch[:-1]).astype(jnp.float32)
    onehot = ((seg[:, None] == jnp.arange(B_GRAPHS, dtype=jnp.int32)[None, :])
              & (n_idx[:, None] < N))
    pmat = onehot.astype(jnp.float32) / jnp.clip(cnt, 1.0)[None, :]

    h0 = jnp.pad(x, ((0, N_PAD - N), (0, 0)))
    h1 = jnp.zeros((N_PAD, HALF), jnp.float32)

    for i in range(5):
        p = params[f"layer{i}"]
        w1 = p["W1"]
        if w1.shape[0] == D_IN:
            w1 = jnp.concatenate([w1, jnp.zeros((D_H - D_IN, D_H), jnp.float32)], axis=0)
        w3 = w1.reshape(2, HALF, D_H)
        z0, z1 = _sc_aggregate(h0, h1, src, dst)
        h1_lin, st = _mlp1(z0, z1, w3, p["b1"].reshape(1, D_H))
        h0, h1 = _mlp2(h1_lin, st,
                       p["gamma"].reshape(1, D_H), p["beta"].reshape(1, D_H),
                       p["W2"], p["b2"].reshape(1, D_H))

    o0, o1 = _pool(h0, h1, pmat)
    return jnp.concatenate([o0, o1], axis=1)


# speedup vs baseline: 1.1594x; 1.1594x over previous
import functools

import jax
import jax.numpy as jnp
from jax import lax
from jax.experimental import pallas as pl
from jax.experimental.pallas import tpu as pltpu
from jax.experimental.pallas import tpu_sc as plsc

N = 10000
N_PAD = 10240
D_IN = 128
D_H = 256
HALF = 128
B_GRAPHS = 64
E = 320000
CHUNK = 64
TILES = 16
CHUNKS_PER_TILE = 320
GROUP = 64
NBUF = 4
E_PAD = TILES * CHUNKS_PER_TILE * CHUNK
ROWS_PER_TILE = N_PAD // TILES
BLK = 256
GRID = N_PAD // BLK


def _sc_aggregate(h0, h1, src, dst):
    mesh = plsc.VectorSubcoreMesh(core_axis_name="c", subcore_axis_name="s")
    out_t = (
        jax.ShapeDtypeStruct((N_PAD, HALF), jnp.float32),
        jax.ShapeDtypeStruct((N_PAD, HALF), jnp.float32),
    )

    @functools.partial(
        pl.kernel,
        mesh=mesh,
        out_type=out_t,
        scratch_types=[
            pltpu.VMEM_SHARED((N_PAD, HALF), jnp.float32),
            pltpu.VMEM((GROUP, CHUNK), jnp.int32),
            pltpu.VMEM((GROUP, CHUNK), jnp.int32),
        ]
        + [pltpu.VMEM((CHUNK, HALF), jnp.float32) for _ in range(NBUF)]
        + [pltpu.SemaphoreType.DMA for _ in range(NBUF)],
    )
    def agg_kernel(h0_hbm, h1_hbm, src_hbm, dst_hbm, z0_hbm, z1_hbm,
                   acc_sh, src_v, dst_v, *bufs_and_sems):
        rows = bufs_and_sems[:NBUF]
        gsem = bufs_and_sems[NBUF:]
        c = lax.axis_index("c")
        s = lax.axis_index("s")
        base = s * ROWS_PER_TILE

        def run(h_hbm, z_hbm):
            pltpu.sync_copy(h_hbm.at[pl.ds(base, ROWS_PER_TILE)],
                            acc_sh.at[pl.ds(base, ROWS_PER_TILE)])
            plsc.subcore_barrier()

            @pl.loop(0, CHUNKS_PER_TILE // GROUP)
            def _(g):
                pltpu.sync_copy(src_hbm.at[s].at[pl.ds(g * GROUP, GROUP)], src_v)
                pltpu.sync_copy(dst_hbm.at[s].at[pl.ds(g * GROUP, GROUP)], dst_v)

                for b in range(NBUF):
                    pltpu.async_copy(h_hbm.at[src_v.at[b]], rows[b], gsem[b])

                @pl.loop(0, GROUP, step=NBUF)
                def _(j):
                    for b in range(NBUF):
                        pltpu.make_async_copy(
                            h_hbm.at[pl.ds(0, CHUNK)], rows[b], gsem[b]).wait()
                        pltpu.sync_copy(rows[b], acc_sh.at[dst_v.at[j + b]],
                                        add=True)

                        @pl.when(j + b + NBUF < GROUP)
                        def _():
                            pltpu.async_copy(h_hbm.at[src_v.at[j + b + NBUF]],
                                             rows[b], gsem[b])

            plsc.subcore_barrier()
            pltpu.sync_copy(acc_sh.at[pl.ds(base, ROWS_PER_TILE)],
                            z_hbm.at[pl.ds(base, ROWS_PER_TILE)])

        @pl.when(c == 0)
        def _():
            run(h0_hbm, z0_hbm)

        @pl.when(c == 1)
        def _():
            run(h1_hbm, z1_hbm)

    return agg_kernel(h0, h1, src, dst)


def _mlp1_body(z0_ref, z1_ref, w_ref, b_ref, h1_ref, st_ref):
    i = pl.program_id(0)
    m = (
        jnp.dot(z0_ref[...], w_ref[0], preferred_element_type=jnp.float32)
        + jnp.dot(z1_ref[...], w_ref[1], preferred_element_type=jnp.float32)
        + b_ref[...]
    )
    h1_ref[...] = m
    rows = i * BLK + lax.broadcasted_iota(jnp.int32, (BLK, 1), 0)
    mm = m * (rows < N).astype(jnp.float32)
    su = jnp.sum(mm, axis=0, keepdims=True)
    sq = jnp.sum(mm * mm, axis=0, keepdims=True)
    upd = jnp.concatenate([su, sq, jnp.zeros((6, D_H), jnp.float32)], axis=0)

    @pl.when(i == 0)
    def _():
        st_ref[...] = jnp.zeros_like(st_ref)

    st_ref[...] += upd


def _mlp1(z0, z1, w3, b1):
    return pl.pallas_call(
        _mlp1_body,
        grid=(GRID,),
        in_specs=[
            pl.BlockSpec((BLK, HALF), lambda i: (i, 0)),
            pl.BlockSpec((BLK, HALF), lambda i: (i, 0)),
            pl.BlockSpec((2, HALF, D_H), lambda i: (0, 0, 0)),
            pl.BlockSpec((1, D_H), lambda i: (0, 0)),
        ],
        out_specs=[
            pl.BlockSpec((BLK, D_H), lambda i: (i, 0)),
            pl.BlockSpec((8, D_H), lambda i: (0, 0)),
        ],
        out_shape=[
            jax.ShapeDtypeStruct((N_PAD, D_H), jnp.float32),
            jax.ShapeDtypeStruct((8, D_H), jnp.float32),
        ],
    )(z0, z1, w3, b1)


def _mlp2_body(h1_ref, st_ref, g_ref, bt_ref, w2_ref, b2_ref, o0_ref, o1_ref):
    st = st_ref[...]
    mean = st[0:1, :] * (1.0 / N)
    var = st[1:2, :] * (1.0 / N) - mean * mean
    inv = lax.rsqrt(var + 1e-5)
    hn = (h1_ref[...] - mean) * (inv * g_ref[...]) + bt_ref[...]
    hn = jnp.maximum(hn, 0.0)
    out = jnp.dot(hn, w2_ref[...], preferred_element_type=jnp.float32) + b2_ref[...]
    out = jnp.maximum(out, 0.0)
    o0_ref[...] = out[:, :HALF]
    o1_ref[...] = out[:, HALF:]


def _mlp2(h1, st, gamma, beta, w2, b2):
    return pl.pallas_call(
        _mlp2_body,
        grid=(GRID,),
        in_specs=[
            pl.BlockSpec((BLK, D_H), lambda i: (i, 0)),
            pl.BlockSpec((8, D_H), lambda i: (0, 0)),
            pl.BlockSpec((1, D_H), lambda i: (0, 0)),
            pl.BlockSpec((1, D_H), lambda i: (0, 0)),
            pl.BlockSpec((D_H, D_H), lambda i: (0, 0)),
            pl.BlockSpec((1, D_H), lambda i: (0, 0)),
        ],
        out_specs=[
            pl.BlockSpec((BLK, HALF), lambda i: (i, 0)),
            pl.BlockSpec((BLK, HALF), lambda i: (i, 0)),
        ],
        out_shape=[
            jax.ShapeDtypeStruct((N_PAD, HALF), jnp.float32),
            jax.ShapeDtypeStruct((N_PAD, HALF), jnp.float32),
        ],
    )(h1, st, gamma, beta, w2, b2)


def _pool_body(h0_ref, h1_ref, p_ref, o0_ref, o1_ref):
    i = pl.program_id(0)

    @pl.when(i == 0)
    def _():
        o0_ref[...] = jnp.zeros_like(o0_ref)
        o1_ref[...] = jnp.zeros_like(o1_ref)

    pt = p_ref[...]
    dn = (((0,), (0,)), ((), ()))
    o0_ref[...] += lax.dot_general(pt, h0_ref[...], dn,
                                   preferred_element_type=jnp.float32)
    o1_ref[...] += lax.dot_general(pt, h1_ref[...], dn,
                                   preferred_element_type=jnp.float32)


def _pool(h0, h1, pmat):
    return pl.pallas_call(
        _pool_body,
        grid=(GRID,),
        in_specs=[
            pl.BlockSpec((BLK, HALF), lambda i: (i, 0)),
            pl.BlockSpec((BLK, HALF), lambda i: (i, 0)),
            pl.BlockSpec((BLK, B_GRAPHS), lambda i: (i, 0)),
        ],
        out_specs=[
            pl.BlockSpec((B_GRAPHS, HALF), lambda i: (0, 0)),
            pl.BlockSpec((B_GRAPHS, HALF), lambda i: (0, 0)),
        ],
        out_shape=[
            jax.ShapeDtypeStruct((B_GRAPHS, HALF), jnp.float32),
            jax.ShapeDtypeStruct((B_GRAPHS, HALF), jnp.float32),
        ],
    )(h0, h1, pmat)


def kernel(x, adj_t, batch, params):
    src = jnp.concatenate([adj_t[0], jnp.zeros((E_PAD - E,), jnp.int32)])
    dst = jnp.concatenate([adj_t[1], jnp.full((E_PAD - E,), N, jnp.int32)])
    src = src.reshape(TILES, CHUNKS_PER_TILE, CHUNK)
    dst = dst.reshape(TILES, CHUNKS_PER_TILE, CHUNK)

    n_idx = jnp.arange(N_PAD, dtype=jnp.int32)
    seg = jnp.clip(jnp.searchsorted(batch, n_idx, side="right") - 1, 0, B_GRAPHS - 1)
    cnt = (batch[1:] - batch[:-1]).astype(jnp.float32)
    onehot = ((seg[:, None] == jnp.arange(B_GRAPHS, dtype=jnp.int32)[None, :])
              & (n_idx[:, None] < N))
    pmat = onehot.astype(jnp.float32) / jnp.clip(cnt, 1.0)[None, :]

    h0 = jnp.pad(x, ((0, N_PAD - N), (0, 0)))
    h1 = jnp.zeros((N_PAD, HALF), jnp.float32)

    for i in range(5):
        p = params[f"layer{i}"]
        w1 = p["W1"]
        if w1.shape[0] == D_IN:
            w1 = jnp.concatenate([w1, jnp.zeros((D_H - D_IN, D_H), jnp.float32)], axis=0)
        w3 = w1.reshape(2, HALF, D_H)
        z0, z1 = _sc_aggregate(h0, h1, src, dst)
        h1_lin, st = _mlp1(z0, z1, w3, p["b1"].reshape(1, D_H))
        h0, h1 = _mlp2(h1_lin, st,
                       p["gamma"].reshape(1, D_H), p["beta"].reshape(1, D_H),
                       p["W2"], p["b2"].reshape(1, D_H))

    o0, o1 = _pool(h0, h1, pmat)
    return jnp.concatenate([o0, o1], axis=1)


# speedup vs baseline: 1.1766x; 1.0148x over previous
import functools

import jax
import jax.numpy as jnp
from jax import lax
from jax.experimental import pallas as pl
from jax.experimental.pallas import tpu as pltpu
from jax.experimental.pallas import tpu_sc as plsc

N = 10000
N_PAD = 10240
D_IN = 128
D_H = 256
HALF = 128
B_GRAPHS = 64
E = 320000
CHUNK = 64
TILES = 16
CHUNKS_PER_TILE = 320
GROUP = 64
NBUF = 4
E_PAD = TILES * CHUNKS_PER_TILE * CHUNK
ROWS_PER_TILE = N_PAD // TILES
BLK = 256
GRID = N_PAD // BLK


def _sc_aggregate(h0, h1, src, dst):
    mesh = plsc.VectorSubcoreMesh(core_axis_name="c", subcore_axis_name="s")
    out_t = (
        jax.ShapeDtypeStruct((N_PAD, HALF), jnp.float32),
        jax.ShapeDtypeStruct((N_PAD, HALF), jnp.float32),
    )

    @functools.partial(
        pl.kernel,
        mesh=mesh,
        out_type=out_t,
        scratch_types=[
            pltpu.VMEM_SHARED((N_PAD, HALF), jnp.float32),
            pltpu.VMEM((GROUP, CHUNK), jnp.int32),
            pltpu.VMEM((GROUP, CHUNK), jnp.int32),
        ]
        + [pltpu.VMEM((CHUNK, HALF), jnp.float32) for _ in range(NBUF)]
        + [pltpu.SemaphoreType.DMA for _ in range(NBUF)],
    )
    def agg_kernel(h0_hbm, h1_hbm, src_hbm, dst_hbm, z0_hbm, z1_hbm,
                   acc_sh, src_v, dst_v, *bufs_and_sems):
        rows = bufs_and_sems[:NBUF]
        gsem = bufs_and_sems[NBUF:]
        c = lax.axis_index("c")
        s = lax.axis_index("s")
        base = s * ROWS_PER_TILE

        def run(h_hbm, z_hbm):
            pltpu.sync_copy(h_hbm.at[pl.ds(base, ROWS_PER_TILE)],
                            acc_sh.at[pl.ds(base, ROWS_PER_TILE)])
            plsc.subcore_barrier()

            @pl.loop(0, CHUNKS_PER_TILE // GROUP)
            def _(g):
                pltpu.sync_copy(src_hbm.at[s].at[pl.ds(g * GROUP, GROUP)], src_v)
                pltpu.sync_copy(dst_hbm.at[s].at[pl.ds(g * GROUP, GROUP)], dst_v)

                for b in range(NBUF):
                    pltpu.async_copy(h_hbm.at[src_v.at[b]], rows[b], gsem[b])

                @pl.loop(0, GROUP, step=NBUF)
                def _(j):
                    for b in range(NBUF):
                        pltpu.make_async_copy(
                            h_hbm.at[pl.ds(0, CHUNK)], rows[b], gsem[b]).wait()
                        pltpu.sync_copy(rows[b], acc_sh.at[dst_v.at[j + b]],
                                        add=True)

                        @pl.when(j + b + NBUF < GROUP)
                        def _():
                            pltpu.async_copy(h_hbm.at[src_v.at[j + b + NBUF]],
                                             rows[b], gsem[b])

            plsc.subcore_barrier()
            pltpu.sync_copy(acc_sh.at[pl.ds(base, ROWS_PER_TILE)],
                            z_hbm.at[pl.ds(base, ROWS_PER_TILE)])

        @pl.when(c == 0)
        def _():
            run(h0_hbm, z0_hbm)

        @pl.when(c == 1)
        def _():
            run(h1_hbm, z1_hbm)

    return agg_kernel(h0, h1, src, dst)


L0_GROUP = 32
L0_CPT = CHUNKS_PER_TILE // 2


def _sc_aggregate_l0(x, zero, src, dst):
    mesh = plsc.VectorSubcoreMesh(core_axis_name="c", subcore_axis_name="s")
    out_t = (
        jax.ShapeDtypeStruct((N_PAD, HALF), jnp.float32),
        jax.ShapeDtypeStruct((N_PAD, HALF), jnp.float32),
    )

    @functools.partial(
        pl.kernel,
        mesh=mesh,
        out_type=out_t,
        scratch_types=[
            pltpu.VMEM_SHARED((N_PAD, HALF), jnp.float32),
            pltpu.VMEM((L0_GROUP, CHUNK), jnp.int32),
            pltpu.VMEM((L0_GROUP, CHUNK), jnp.int32),
        ]
        + [pltpu.VMEM((CHUNK, HALF), jnp.float32) for _ in range(NBUF)]
        + [pltpu.SemaphoreType.DMA for _ in range(NBUF)],
    )
    def agg_kernel(x_hbm, zero_hbm, src_hbm, dst_hbm, z0_hbm, z1_hbm,
                   acc_sh, src_v, dst_v, *bufs_and_sems):
        rows = bufs_and_sems[:NBUF]
        gsem = bufs_and_sems[NBUF:]
        c = lax.axis_index("c")
        s = lax.axis_index("s")
        base = s * ROWS_PER_TILE

        def run(init_hbm, z_hbm, chunk0):
            pltpu.sync_copy(init_hbm.at[pl.ds(base, ROWS_PER_TILE)],
                            acc_sh.at[pl.ds(base, ROWS_PER_TILE)])
            plsc.subcore_barrier()

            @pl.loop(0, L0_CPT // L0_GROUP)
            def _(g):
                pltpu.sync_copy(
                    src_hbm.at[s].at[pl.ds(chunk0 + g * L0_GROUP, L0_GROUP)], src_v)
                pltpu.sync_copy(
                    dst_hbm.at[s].at[pl.ds(chunk0 + g * L0_GROUP, L0_GROUP)], dst_v)

                for b in range(NBUF):
                    pltpu.async_copy(x_hbm.at[src_v.at[b]], rows[b], gsem[b])

                @pl.loop(0, L0_GROUP, step=NBUF)
                def _(j):
                    for b in range(NBUF):
                        pltpu.make_async_copy(
                            x_hbm.at[pl.ds(0, CHUNK)], rows[b], gsem[b]).wait()
                        pltpu.sync_copy(rows[b], acc_sh.at[dst_v.at[j + b]],
                                        add=True)

                        @pl.when(j + b + NBUF < L0_GROUP)
                        def _():
                            pltpu.async_copy(
                                x_hbm.at[src_v.at[j + b + NBUF]], rows[b], gsem[b])

            plsc.subcore_barrier()
            pltpu.sync_copy(acc_sh.at[pl.ds(base, ROWS_PER_TILE)],
                            z_hbm.at[pl.ds(base, ROWS_PER_TILE)])

        @pl.when(c == 0)
        def _():
            run(x_hbm, z0_hbm, 0)

        @pl.when(c == 1)
        def _():
            run(zero_hbm, z1_hbm, L0_CPT)

    return agg_kernel(x, zero, src, dst)


def _mlp1_body(z0_ref, z1_ref, w_ref, b_ref, h1_ref, st_ref):
    i = pl.program_id(0)
    m = (
        jnp.dot(z0_ref[...], w_ref[0], preferred_element_type=jnp.float32, precision=lax.Precision.HIGHEST)
        + jnp.dot(z1_ref[...], w_ref[1], preferred_element_type=jnp.float32, precision=lax.Precision.HIGHEST)
        + b_ref[...]
    )
    h1_ref[...] = m
    rows = i * BLK + lax.broadcasted_iota(jnp.int32, (BLK, 1), 0)
    mm = m * (rows < N).astype(jnp.float32)
    su = jnp.sum(mm, axis=0, keepdims=True)
    sq = jnp.sum(mm * mm, axis=0, keepdims=True)
    upd = jnp.concatenate([su, sq, jnp.zeros((6, D_H), jnp.float32)], axis=0)

    @pl.when(i == 0)
    def _():
        st_ref[...] = jnp.zeros_like(st_ref)

    st_ref[...] += upd


def _mlp1(z0, z1, w3, b1):
    return pl.pallas_call(
        _mlp1_body,
        grid=(GRID,),
        in_specs=[
            pl.BlockSpec((BLK, HALF), lambda i: (i, 0)),
            pl.BlockSpec((BLK, HALF), lambda i: (i, 0)),
            pl.BlockSpec((2, HALF, D_H), lambda i: (0, 0, 0)),
            pl.BlockSpec((1, D_H), lambda i: (0, 0)),
        ],
        out_specs=[
            pl.BlockSpec((BLK, D_H), lambda i: (i, 0)),
            pl.BlockSpec((8, D_H), lambda i: (0, 0)),
        ],
        out_shape=[
            jax.ShapeDtypeStruct((N_PAD, D_H), jnp.float32),
            jax.ShapeDtypeStruct((8, D_H), jnp.float32),
        ],
    )(z0, z1, w3, b1)


def _mlp2_body(h1_ref, st_ref, g_ref, bt_ref, w2_ref, b2_ref, o0_ref, o1_ref):
    st = st_ref[...]
    mean = st[0:1, :] * (1.0 / N)
    var = st[1:2, :] * (1.0 / N) - mean * mean
    inv = lax.rsqrt(var + 1e-5)
    hn = (h1_ref[...] - mean) * (inv * g_ref[...]) + bt_ref[...]
    hn = jnp.maximum(hn, 0.0)
    out = jnp.dot(hn, w2_ref[...], preferred_element_type=jnp.float32, precision=lax.Precision.HIGHEST) + b2_ref[...]
    out = jnp.maximum(out, 0.0)
    o0_ref[...] = out[:, :HALF]
    o1_ref[...] = out[:, HALF:]


def _mlp2(h1, st, gamma, beta, w2, b2):
    return pl.pallas_call(
        _mlp2_body,
        grid=(GRID,),
        in_specs=[
            pl.BlockSpec((BLK, D_H), lambda i: (i, 0)),
            pl.BlockSpec((8, D_H), lambda i: (0, 0)),
            pl.BlockSpec((1, D_H), lambda i: (0, 0)),
            pl.BlockSpec((1, D_H), lambda i: (0, 0)),
            pl.BlockSpec((D_H, D_H), lambda i: (0, 0)),
            pl.BlockSpec((1, D_H), lambda i: (0, 0)),
        ],
        out_specs=[
            pl.BlockSpec((BLK, HALF), lambda i: (i, 0)),
            pl.BlockSpec((BLK, HALF), lambda i: (i, 0)),
        ],
        out_shape=[
            jax.ShapeDtypeStruct((N_PAD, HALF), jnp.float32),
            jax.ShapeDtypeStruct((N_PAD, HALF), jnp.float32),
        ],
    )(h1, st, gamma, beta, w2, b2)


def _pool_body(h0_ref, h1_ref, p_ref, o0_ref, o1_ref):
    i = pl.program_id(0)

    @pl.when(i == 0)
    def _():
        o0_ref[...] = jnp.zeros_like(o0_ref)
        o1_ref[...] = jnp.zeros_like(o1_ref)

    pt = p_ref[...]
    dn = (((0,), (0,)), ((), ()))
    o0_ref[...] += lax.dot_general(pt, h0_ref[...], dn,
                                   preferred_element_type=jnp.float32,
                                   precision=lax.Precision.HIGHEST)
    o1_ref[...] += lax.dot_general(pt, h1_ref[...], dn,
                                   preferred_element_type=jnp.float32,
                                   precision=lax.Precision.HIGHEST)


def _pool(h0, h1, pmat):
    return pl.pallas_call(
        _pool_body,
        grid=(GRID,),
        in_specs=[
            pl.BlockSpec((BLK, HALF), lambda i: (i, 0)),
            pl.BlockSpec((BLK, HALF), lambda i: (i, 0)),
            pl.BlockSpec((BLK, B_GRAPHS), lambda i: (i, 0)),
        ],
        out_specs=[
            pl.BlockSpec((B_GRAPHS, HALF), lambda i: (0, 0)),
            pl.BlockSpec((B_GRAPHS, HALF), lambda i: (0, 0)),
        ],
        out_shape=[
            jax.ShapeDtypeStruct((B_GRAPHS, HALF), jnp.float32),
            jax.ShapeDtypeStruct((B_GRAPHS, HALF), jnp.float32),
        ],
    )(h0, h1, pmat)


def kernel(x, adj_t, batch, params):
    src = jnp.concatenate([adj_t[0], jnp.zeros((E_PAD - E,), jnp.int32)])
    dst = jnp.concatenate([adj_t[1], jnp.full((E_PAD - E,), N, jnp.int32)])
    src = src.reshape(TILES, CHUNKS_PER_TILE, CHUNK)
    dst = dst.reshape(TILES, CHUNKS_PER_TILE, CHUNK)

    n_idx = jnp.arange(N_PAD, dtype=jnp.int32)
    seg = jnp.clip(jnp.searchsorted(batch, n_idx, side="right") - 1, 0, B_GRAPHS - 1)
    cnt = (batch[1:] - batch[:-1]).astype(jnp.float32)
    onehot = ((seg[:, None] == jnp.arange(B_GRAPHS, dtype=jnp.int32)[None, :])
              & (n_idx[:, None] < N))
    pmat = onehot.astype(jnp.float32) / jnp.clip(cnt, 1.0)[None, :]

    xp = jnp.pad(x, ((0, N_PAD - N), (0, 0)))
    zero = jnp.zeros((N_PAD, HALF), jnp.float32)
    h0, h1 = xp, zero

    for i in range(5):
        p = params[f"layer{i}"]
        w1 = p["W1"]
        if i == 0:
            w3 = jnp.stack([w1, w1])
            z0, z1 = _sc_aggregate_l0(h0, h1, src, dst)
        else:
            w3 = w1.reshape(2, HALF, D_H)
            z0, z1 = _sc_aggregate(h0, h1, src, dst)
        h1_lin, st = _mlp1(z0, z1, w3, p["b1"].reshape(1, D_H))
        h0, h1 = _mlp2(h1_lin, st,
                       p["gamma"].reshape(1, D_H), p["beta"].reshape(1, D_H),
                       p["W2"], p["b2"].reshape(1, D_H))

    o0, o1 = _pool(h0, h1, pmat)
    return jnp.concatenate([o0, o1], axis=1)


# speedup vs baseline: 1.1770x; 1.0004x over previous
import functools

import jax
import jax.numpy as jnp
from jax import lax
from jax.experimental import pallas as pl
from jax.experimental.pallas import tpu as pltpu
from jax.experimental.pallas import tpu_sc as plsc

N = 10000
N_PAD = 10240
D_IN = 128
D_H = 256
HALF = 128
B_GRAPHS = 64
E = 320000
CHUNK = 64
TILES = 16
CHUNKS_PER_TILE = 320
GROUP = 64
NBUF = 4
E_PAD = TILES * CHUNKS_PER_TILE * CHUNK
ROWS_PER_TILE = N_PAD // TILES
BLK = 256
GRID = N_PAD // BLK


def _sc_aggregate(h0, h1, src, dst):
    mesh = plsc.VectorSubcoreMesh(core_axis_name="c", subcore_axis_name="s")
    out_t = (
        jax.ShapeDtypeStruct((N_PAD, HALF), jnp.float32),
        jax.ShapeDtypeStruct((N_PAD, HALF), jnp.float32),
    )

    @functools.partial(
        pl.kernel,
        mesh=mesh,
        out_type=out_t,
        scratch_types=[
            pltpu.VMEM_SHARED((N_PAD, HALF), jnp.float32),
            pltpu.VMEM((GROUP, CHUNK), jnp.int32),
            pltpu.VMEM((GROUP, CHUNK), jnp.int32),
        ]
        + [pltpu.VMEM((CHUNK, HALF), jnp.float32) for _ in range(NBUF)]
        + [pltpu.SemaphoreType.DMA for _ in range(NBUF)],
    )
    def agg_kernel(h0_hbm, h1_hbm, src_hbm, dst_hbm, z0_hbm, z1_hbm,
                   acc_sh, src_v, dst_v, *bufs_and_sems):
        rows = bufs_and_sems[:NBUF]
        gsem = bufs_and_sems[NBUF:]
        c = lax.axis_index("c")
        s = lax.axis_index("s")
        base = s * ROWS_PER_TILE

        def run(h_hbm, z_hbm):
            pltpu.sync_copy(h_hbm.at[pl.ds(base, ROWS_PER_TILE)],
                            acc_sh.at[pl.ds(base, ROWS_PER_TILE)])
            plsc.subcore_barrier()

            @pl.loop(0, CHUNKS_PER_TILE // GROUP)
            def _(g):
                pltpu.sync_copy(src_hbm.at[s].at[pl.ds(g * GROUP, GROUP)], src_v)
                pltpu.sync_copy(dst_hbm.at[s].at[pl.ds(g * GROUP, GROUP)], dst_v)

                for b in range(NBUF):
                    pltpu.async_copy(h_hbm.at[src_v.at[b]], rows[b], gsem[b])

                @pl.loop(0, GROUP, step=NBUF)
                def _(j):
                    for b in range(NBUF):
                        pltpu.make_async_copy(
                            h_hbm.at[pl.ds(0, CHUNK)], rows[b], gsem[b]).wait()
                        pltpu.sync_copy(rows[b], acc_sh.at[dst_v.at[j + b]],
                                        add=True)

                        @pl.when(j + b + NBUF < GROUP)
                        def _():
                            pltpu.async_copy(h_hbm.at[src_v.at[j + b + NBUF]],
                                             rows[b], gsem[b])

            plsc.subcore_barrier()
            pltpu.sync_copy(acc_sh.at[pl.ds(base, ROWS_PER_TILE)],
                            z_hbm.at[pl.ds(base, ROWS_PER_TILE)])

        @pl.when(c == 0)
        def _():
            run(h0_hbm, z0_hbm)

        @pl.when(c == 1)
        def _():
            run(h1_hbm, z1_hbm)

    return agg_kernel(h0, h1, src, dst)


L0_GROUP = 32
L0_CPT = CHUNKS_PER_TILE // 2


def _sc_aggregate_l0(x, zero, src, dst):
    mesh = plsc.VectorSubcoreMesh(core_axis_name="c", subcore_axis_name="s")
    out_t = (
        jax.ShapeDtypeStruct((N_PAD, HALF), jnp.float32),
        jax.ShapeDtypeStruct((N_PAD, HALF), jnp.float32),
    )

    @functools.partial(
        pl.kernel,
        mesh=mesh,
        out_type=out_t,
        scratch_types=[
            pltpu.VMEM_SHARED((N_PAD, HALF), jnp.float32),
            pltpu.VMEM((L0_GROUP, CHUNK), jnp.int32),
            pltpu.VMEM((L0_GROUP, CHUNK), jnp.int32),
        ]
        + [pltpu.VMEM((CHUNK, HALF), jnp.float32) for _ in range(NBUF)]
        + [pltpu.SemaphoreType.DMA for _ in range(NBUF)],
    )
    def agg_kernel(x_hbm, zero_hbm, src_hbm, dst_hbm, z0_hbm, z1_hbm,
                   acc_sh, src_v, dst_v, *bufs_and_sems):
        rows = bufs_and_sems[:NBUF]
        gsem = bufs_and_sems[NBUF:]
        c = lax.axis_index("c")
        s = lax.axis_index("s")
        base = s * ROWS_PER_TILE

        def run(init_hbm, z_hbm, chunk0):
            pltpu.sync_copy(init_hbm.at[pl.ds(base, ROWS_PER_TILE)],
                            acc_sh.at[pl.ds(base, ROWS_PER_TILE)])
            plsc.subcore_barrier()

            @pl.loop(0, L0_CPT // L0_GROUP)
            def _(g):
                pltpu.sync_copy(
                    src_hbm.at[s].at[pl.ds(chunk0 + g * L0_GROUP, L0_GROUP)], src_v)
                pltpu.sync_copy(
                    dst_hbm.at[s].at[pl.ds(chunk0 + g * L0_GROUP, L0_GROUP)], dst_v)

                for b in range(NBUF):
                    pltpu.async_copy(x_hbm.at[src_v.at[b]], rows[b], gsem[b])

                @pl.loop(0, L0_GROUP, step=NBUF)
                def _(j):
                    for b in range(NBUF):
                        pltpu.make_async_copy(
                            x_hbm.at[pl.ds(0, CHUNK)], rows[b], gsem[b]).wait()
                        pltpu.sync_copy(rows[b], acc_sh.at[dst_v.at[j + b]],
                                        add=True)

                        @pl.when(j + b + NBUF < L0_GROUP)
                        def _():
                            pltpu.async_copy(
                                x_hbm.at[src_v.at[j + b + NBUF]], rows[b], gsem[b])

            plsc.subcore_barrier()
            pltpu.sync_copy(acc_sh.at[pl.ds(base, ROWS_PER_TILE)],
                            z_hbm.at[pl.ds(base, ROWS_PER_TILE)])

        @pl.when(c == 0)
        def _():
            run(x_hbm, z0_hbm, 0)

        @pl.when(c == 1)
        def _():
            run(zero_hbm, z1_hbm, L0_CPT)

    return agg_kernel(x, zero, src, dst)


def _mlp1_body(z0_ref, z1_ref, w_ref, b_ref, h1_ref, st_ref):
    i = pl.program_id(0)
    m = (
        jnp.dot(z0_ref[...], w_ref[0], preferred_element_type=jnp.float32, precision=lax.Precision.HIGHEST)
        + jnp.dot(z1_ref[...], w_ref[1], preferred_element_type=jnp.float32, precision=lax.Precision.HIGHEST)
        + b_ref[...]
    )
    h1_ref[...] = m
    rows = i * BLK + lax.broadcasted_iota(jnp.int32, (BLK, 1), 0)
    mm = m * (rows < N).astype(jnp.float32)
    su = jnp.sum(mm, axis=0, keepdims=True)
    sq = jnp.sum(mm * mm, axis=0, keepdims=True)
    upd = jnp.concatenate([su, sq, jnp.zeros((6, D_H), jnp.float32)], axis=0)

    @pl.when(i == 0)
    def _():
        st_ref[...] = jnp.zeros_like(st_ref)

    st_ref[...] += upd


def _mlp1(z0, z1, w3, b1):
    return pl.pallas_call(
        _mlp1_body,
        grid=(GRID,),
        in_specs=[
            pl.BlockSpec((BLK, HALF), lambda i: (i, 0)),
            pl.BlockSpec((BLK, HALF), lambda i: (i, 0)),
            pl.BlockSpec((2, HALF, D_H), lambda i: (0, 0, 0)),
            pl.BlockSpec((1, D_H), lambda i: (0, 0)),
        ],
        out_specs=[
            pl.BlockSpec((BLK, D_H), lambda i: (i, 0)),
            pl.BlockSpec((8, D_H), lambda i: (0, 0)),
        ],
        out_shape=[
            jax.ShapeDtypeStruct((N_PAD, D_H), jnp.float32),
            jax.ShapeDtypeStruct((8, D_H), jnp.float32),
        ],
    )(z0, z1, w3, b1)


def _mlp2_body(h1_ref, st_ref, g_ref, bt_ref, w2_ref, b2_ref, o0_ref, o1_ref):
    st = st_ref[...]
    mean = st[0:1, :] * (1.0 / N)
    var = st[1:2, :] * (1.0 / N) - mean * mean
    inv = lax.rsqrt(var + 1e-5)
    hn = (h1_ref[...] - mean) * (inv * g_ref[...]) + bt_ref[...]
    hn = jnp.maximum(hn, 0.0)
    out = jnp.dot(hn, w2_ref[...], preferred_element_type=jnp.float32, precision=lax.Precision.HIGHEST) + b2_ref[...]
    out = jnp.maximum(out, 0.0)
    o0_ref[...] = out[:, :HALF]
    o1_ref[...] = out[:, HALF:]


def _mlp2(h1, st, gamma, beta, w2, b2):
    return pl.pallas_call(
        _mlp2_body,
        grid=(GRID,),
        in_specs=[
            pl.BlockSpec((BLK, D_H), lambda i: (i, 0)),
            pl.BlockSpec((8, D_H), lambda i: (0, 0)),
            pl.BlockSpec((1, D_H), lambda i: (0, 0)),
            pl.BlockSpec((1, D_H), lambda i: (0, 0)),
            pl.BlockSpec((D_H, D_H), lambda i: (0, 0)),
            pl.BlockSpec((1, D_H), lambda i: (0, 0)),
        ],
        out_specs=[
            pl.BlockSpec((BLK, HALF), lambda i: (i, 0)),
            pl.BlockSpec((BLK, HALF), lambda i: (i, 0)),
        ],
        out_shape=[
            jax.ShapeDtypeStruct((N_PAD, HALF), jnp.float32),
            jax.ShapeDtypeStruct((N_PAD, HALF), jnp.float32),
        ],
    )(h1, st, gamma, beta, w2, b2)


def _pool_body(h0_ref, h1_ref, p_ref, o0_ref, o1_ref):
    i = pl.program_id(0)

    @pl.when(i == 0)
    def _():
        o0_ref[...] = jnp.zeros_like(o0_ref)
        o1_ref[...] = jnp.zeros_like(o1_ref)

    pt = p_ref[...]
    dn = (((0,), (0,)), ((), ()))
    o0_ref[...] += lax.dot_general(pt, h0_ref[...], dn,
                                   preferred_element_type=jnp.float32,
                                   precision=lax.Precision.HIGHEST)
    o1_ref[...] += lax.dot_general(pt, h1_ref[...], dn,
                                   preferred_element_type=jnp.float32,
                                   precision=lax.Precision.HIGHEST)


def _pool(h0, h1, pmat):
    return pl.pallas_call(
        _pool_body,
        grid=(GRID,),
        in_specs=[
            pl.BlockSpec((BLK, HALF), lambda i: (i, 0)),
            pl.BlockSpec((BLK, HALF), lambda i: (i, 0)),
            pl.BlockSpec((BLK, B_GRAPHS), lambda i: (i, 0)),
        ],
        out_specs=[
            pl.BlockSpec((B_GRAPHS, HALF), lambda i: (0, 0)),
            pl.BlockSpec((B_GRAPHS, HALF), lambda i: (0, 0)),
        ],
        out_shape=[
            jax.ShapeDtypeStruct((B_GRAPHS, HALF), jnp.float32),
            jax.ShapeDtypeStruct((B_GRAPHS, HALF), jnp.float32),
        ],
    )(h0, h1, pmat)


def kernel(x, adj_t, batch, params):
    pad_dst = (N + jnp.arange(E_PAD - E, dtype=jnp.int32) % (N_PAD - N))
    src = jnp.concatenate([adj_t[0], jnp.zeros((E_PAD - E,), jnp.int32)])
    dst = jnp.concatenate([adj_t[1], pad_dst])
    src = src.reshape(TILES, CHUNKS_PER_TILE, CHUNK)
    dst = dst.reshape(TILES, CHUNKS_PER_TILE, CHUNK)

    n_idx = jnp.arange(N_PAD, dtype=jnp.int32)
    seg = jnp.clip(jnp.searchsorted(batch, n_idx, side="right") - 1, 0, B_GRAPHS - 1)
    cnt = (batch[1:] - batch[:-1]).astype(jnp.float32)
    onehot = ((seg[:, None] == jnp.arange(B_GRAPHS, dtype=jnp.int32)[None, :])
              & (n_idx[:, None] < N))
    pmat = onehot.astype(jnp.float32) / jnp.clip(cnt, 1.0)[None, :]

    xp = jnp.pad(x, ((0, N_PAD - N), (0, 0)))
    zero = jnp.zeros((N_PAD, HALF), jnp.float32)
    h0, h1 = xp, zero

    for i in range(5):
        p = params[f"layer{i}"]
        w1 = p["W1"]
        if i == 0:
            w3 = jnp.stack([w1, w1])
            z0, z1 = _sc_aggregate_l0(h0, h1, src, dst)
        else:
            w3 = w1.reshape(2, HALF, D_H)
            z0, z1 = _sc_aggregate(h0, h1, src, dst)
        h1_lin, st = _mlp1(z0, z1, w3, p["b1"].reshape(1, D_H))
        h0, h1 = _mlp2(h1_lin, st,
                       p["gamma"].reshape(1, D_H), p["beta"].reshape(1, D_H),
                       p["W2"], p["b2"].reshape(1, D_H))

    o0, o1 = _pool(h0, h1, pmat)
    return jnp.concatenate([o0, o1], axis=1)
